# Initial kernel scaffold; baseline (speedup 1.0000x reference)
import jax
import jax.numpy as jnp
from jax.experimental import pallas as pl


def kernel(X, table):
    raise NotImplementedError("write your pallas kernel here")


# speedup vs baseline: 2.5629x; 2.5629x over previous
import functools

import jax
import jax.numpy as jnp
from jax import lax
from jax.experimental import pallas as pl
from jax.experimental.pallas import tpu as pltpu
from jax.experimental.pallas import tpu_sc as plsc

ROWS, COLS = 16384, 200
VOCAB, DIM = 64, 64
B = ROWS * COLS
NW = 32
B_PER_W = B // NW
CHUNK = 800
N_CHUNKS = B_PER_W // CHUNK
N_PAIRS = N_CHUNKS // 2


def _make_kernel():
    mesh = plsc.VectorSubcoreMesh(core_axis_name="c", subcore_axis_name="s")

    @functools.partial(
        pl.kernel,
        mesh=mesh,
        out_type=jax.ShapeDtypeStruct((B, DIM), jnp.float32),
        scratch_types=[
            pltpu.VMEM((CHUNK,), jnp.int32),
            pltpu.VMEM((CHUNK,), jnp.int32),
            pltpu.VMEM((CHUNK, DIM), jnp.float32),
            pltpu.VMEM((CHUNK, DIM), jnp.float32),
            pltpu.SemaphoreType.DMA,
            pltpu.SemaphoreType.DMA,
            pltpu.SemaphoreType.DMA,
            pltpu.SemaphoreType.DMA,
            pltpu.SemaphoreType.DMA,
            pltpu.SemaphoreType.DMA,
        ],
        compiler_params=pltpu.CompilerParams(use_tc_tiling_on_sc=False),
    )
    def gather_kernel(idx_hbm, table_hbm, out_hbm,
                      idx0, idx1, rows0, rows1,
                      si0, si1, sg0, sg1, so0, so1):
        wid = lax.axis_index("s") * 2 + lax.axis_index("c")
        w_base = wid * B_PER_W
        idx_v = (idx0, idx1)
        rows_v = (rows0, rows1)
        sem_i = (si0, si1)
        sem_g = (sg0, sg1)
        sem_o = (so0, so1)

        for b in range(2):
            pltpu.async_copy(
                idx_hbm.at[pl.ds(w_base + b * CHUNK, CHUNK)], idx_v[b],
                sem_i[b])

        def body(j, _):
            for b in range(2):
                base = w_base + (2 * j + b) * CHUNK
                pltpu.make_async_copy(
                    idx_hbm.at[pl.ds(w_base, CHUNK)], idx_v[b],
                    sem_i[b]).wait()

                @pl.when(j >= 1)
                def _():
                    pltpu.make_async_copy(
                        rows_v[b], out_hbm.at[pl.ds(w_base, CHUNK)],
                        sem_o[b]).wait()

                pltpu.async_copy(
                    table_hbm.at[idx_v[b]], rows_v[b], sem_g[b]).wait()

                @pl.when(j < N_PAIRS - 1)
                def _():
                    pltpu.async_copy(
                        idx_hbm.at[pl.ds(base + 2 * CHUNK, CHUNK)],
                        idx_v[b], sem_i[b])

                pltpu.async_copy(
                    rows_v[b], out_hbm.at[pl.ds(base, CHUNK)], sem_o[b])
            return 0

        lax.fori_loop(0, N_PAIRS, body, 0)

        for b in range(2):
            pltpu.make_async_copy(
                rows_v[b], out_hbm.at[pl.ds(w_base, CHUNK)],
                sem_o[b]).wait()

    return gather_kernel


_gather = _make_kernel()


@jax.jit
def kernel(X, table):
    idx = X.reshape(B)
    flat = _gather(idx, table)
    return flat.reshape(ROWS, COLS, DIM)


# speedup vs baseline: 5.7977x; 2.2622x over previous
import functools

import jax
import jax.numpy as jnp
from jax import lax
from jax.experimental import pallas as pl
from jax.experimental.pallas import tpu as pltpu
from jax.experimental.pallas import tpu_sc as plsc

ROWS, COLS = 16384, 200
VOCAB, DIM = 64, 64
B = ROWS * COLS
NW = 32
B_PER_W = B // NW
CHUNK = 800
N_CHUNKS = B_PER_W // CHUNK
N_PAIRS = N_CHUNKS // 2


def _make_kernel():
    mesh = plsc.VectorSubcoreMesh(core_axis_name="c", subcore_axis_name="s")

    @functools.partial(
        pl.kernel,
        mesh=mesh,
        out_type=jax.ShapeDtypeStruct((B, DIM), jnp.float32),
        scratch_types=[
            pltpu.VMEM((CHUNK,), jnp.int32),
            pltpu.VMEM((CHUNK,), jnp.int32),
            pltpu.VMEM((CHUNK, DIM), jnp.float32),
            pltpu.VMEM((CHUNK, DIM), jnp.float32),
            pltpu.VMEM_SHARED((VOCAB, DIM), jnp.float32),
            pltpu.SemaphoreType.DMA,
            pltpu.SemaphoreType.DMA,
            pltpu.SemaphoreType.DMA,
            pltpu.SemaphoreType.DMA,
            pltpu.SemaphoreType.DMA,
            pltpu.SemaphoreType.DMA,
        ],
        compiler_params=pltpu.CompilerParams(use_tc_tiling_on_sc=False),
    )
    def gather_kernel(idx_hbm, table_hbm, out_hbm,
                      idx0, idx1, rows0, rows1, table_v,
                      si0, si1, sg0, sg1, so0, so1):
        wid = lax.axis_index("s") * 2 + lax.axis_index("c")
        w_base = wid * B_PER_W
        pltpu.sync_copy(table_hbm, table_v)
        idx_v = (idx0, idx1)
        rows_v = (rows0, rows1)
        sem_i = (si0, si1)
        sem_g = (sg0, sg1)
        sem_o = (so0, so1)

        for b in range(2):
            pltpu.async_copy(
                idx_hbm.at[pl.ds(w_base + b * CHUNK, CHUNK)], idx_v[b],
                sem_i[b])

        def body(j, _):
            for b in range(2):
                base = w_base + (2 * j + b) * CHUNK
                pltpu.make_async_copy(
                    idx_hbm.at[pl.ds(w_base, CHUNK)], idx_v[b],
                    sem_i[b]).wait()

                @pl.when(j >= 1)
                def _():
                    pltpu.make_async_copy(
                        rows_v[b], out_hbm.at[pl.ds(w_base, CHUNK)],
                        sem_o[b]).wait()

                pltpu.async_copy(
                    table_v.at[idx_v[b]], rows_v[b], sem_g[b]).wait()

                @pl.when(j < N_PAIRS - 1)
                def _():
                    pltpu.async_copy(
                        idx_hbm.at[pl.ds(base + 2 * CHUNK, CHUNK)],
                        idx_v[b], sem_i[b])

                pltpu.async_copy(
                    rows_v[b], out_hbm.at[pl.ds(base, CHUNK)], sem_o[b])
            return 0

        lax.fori_loop(0, N_PAIRS, body, 0)

        for b in range(2):
            pltpu.make_async_copy(
                rows_v[b], out_hbm.at[pl.ds(w_base, CHUNK)],
                sem_o[b]).wait()

    return gather_kernel


_gather = _make_kernel()


@jax.jit
def kernel(X, table):
    idx = X.reshape(B)
    flat = _gather(idx, table)
    return flat.reshape(ROWS, COLS, DIM)
